# baseline (speedup 1.0000x reference)
import functools

import jax
import jax.numpy as jnp
from jax import lax
from jax.experimental import pallas as pl
from jax.experimental.pallas import tpu as pltpu
from jax.experimental.pallas import tpu_sc as plsc

N_NODES = 10000
HIDDEN = 128
PRED_CH = 64
N_EDGES = 320000

NC = 2
NS = 16
NW = NC * NS
CHUNK = 128
C_A = 64
C_B = 96
C_PAIR = C_A + C_B
C_MAX = max(C_A, C_B)
TOT_CHUNKS = NS * C_PAIR + (C_MAX - C_B)
E_PAD = TOT_CHUNKS * CHUNK
N_ACC = 10240
ROWS_PER_TILE = N_ACC // NS

_sc_mesh = plsc.VectorSubcoreMesh(core_axis_name="c", subcore_axis_name="s")


@functools.partial(
    pl.kernel,
    out_type=jax.ShapeDtypeStruct((NC, N_ACC, HIDDEN), jnp.float32),
    mesh=_sc_mesh,
    scratch_types=[
        pltpu.VMEM((C_MAX, CHUNK), jnp.int32),
        pltpu.VMEM((C_MAX, CHUNK), jnp.int32),
        pltpu.VMEM((CHUNK, HIDDEN), jnp.float32),
        pltpu.VMEM_SHARED((N_ACC, HIDDEN), jnp.float32),
        pltpu.SemaphoreType.DMA,
    ],
)
def _sc_segment_sum(x_hbm, src_hbm, dst_hbm, zeros_hbm, out_hbm,
                    src_v, dst_v, rows_v, acc, sem):
    cid = lax.axis_index("c")
    sid = lax.axis_index("s")
    base = sid * C_PAIR + cid * C_A
    n_chunks = jnp.where(cid == 0, C_A, C_B)
    pltpu.sync_copy(zeros_hbm, acc.at[pl.ds(sid * ROWS_PER_TILE, ROWS_PER_TILE)])
    pltpu.sync_copy(src_hbm.at[pl.ds(base, C_MAX)], src_v)
    pltpu.sync_copy(dst_hbm.at[pl.ds(base, C_MAX)], dst_v)
    plsc.subcore_barrier()

    def body(j, carry):
        pltpu.async_copy(x_hbm.at[src_v.at[j]], rows_v, sem).wait()
        pltpu.sync_copy(rows_v, acc.at[dst_v.at[j]], add=True)
        return carry

    lax.fori_loop(0, n_chunks, body, 0)
    plsc.subcore_barrier()
    pltpu.sync_copy(acc.at[pl.ds(sid * ROWS_PER_TILE, ROWS_PER_TILE)],
                    out_hbm.at[cid, pl.ds(sid * ROWS_PER_TILE, ROWS_PER_TILE)])


BLK = 1000


def _tc_body(p_ref, u_ref, Wh_ref, Wu_ref, b_ref, Wp_ref, bp_ref, xn_ref, y_ref):
    agg = p_ref[0] + p_ref[1]
    h = jnp.dot(agg, Wh_ref[...], preferred_element_type=jnp.float32)
    h = h + jnp.dot(u_ref[...], Wu_ref[...], preferred_element_type=jnp.float32)
    h = h + b_ref[...]
    xn = jnp.maximum(h, 0.0)
    xn_ref[...] = xn
    y_ref[...] = jnp.dot(xn, Wp_ref[...], preferred_element_type=jnp.float32) + bp_ref[...]


_tc_update = pl.pallas_call(
    _tc_body,
    grid=(N_NODES // BLK,),
    in_specs=[
        pl.BlockSpec((NC, BLK, HIDDEN), lambda i: (0, i, 0)),
        pl.BlockSpec((BLK, HIDDEN), lambda i: (i, 0)),
        pl.BlockSpec((HIDDEN, HIDDEN), lambda i: (0, 0)),
        pl.BlockSpec((HIDDEN, HIDDEN), lambda i: (0, 0)),
        pl.BlockSpec((1, HIDDEN), lambda i: (0, 0)),
        pl.BlockSpec((HIDDEN, PRED_CH), lambda i: (0, 0)),
        pl.BlockSpec((1, PRED_CH), lambda i: (0, 0)),
    ],
    out_specs=[
        pl.BlockSpec((BLK, HIDDEN), lambda i: (i, 0)),
        pl.BlockSpec((BLK, PRED_CH), lambda i: (i, 0)),
    ],
    out_shape=[
        jax.ShapeDtypeStruct((N_NODES, HIDDEN), jnp.float32),
        jax.ShapeDtypeStruct((N_NODES, PRED_CH), jnp.float32),
    ],
)


def kernel(x, u, edge_index, W_h, W_u, b, W_p, b_p):
    src = edge_index[0].astype(jnp.int32)
    dst = edge_index[1].astype(jnp.int32)
    pad = E_PAD - N_EDGES
    src = jnp.concatenate([src, jnp.full((pad,), N_NODES, jnp.int32)])
    dst = jnp.concatenate([dst, jnp.zeros((pad,), jnp.int32)])
    src3 = src.reshape(TOT_CHUNKS, CHUNK)
    dst3 = dst.reshape(TOT_CHUNKS, CHUNK)
    x_pad = jnp.concatenate([x, jnp.zeros((1, HIDDEN), x.dtype)], axis=0)
    zeros_blk = jnp.zeros((ROWS_PER_TILE, HIDDEN), jnp.float32)

    partial = _sc_segment_sum(x_pad, src3, dst3, zeros_blk)

    x_next, y = _tc_update(partial, u, W_h, W_u, b.reshape(1, HIDDEN),
                           W_p, b_p.reshape(1, PRED_CH))
    return (x_next, y)


# speedup vs baseline: 1.5896x; 1.5896x over previous
import functools

import jax
import jax.numpy as jnp
from jax import lax
from jax.experimental import pallas as pl
from jax.experimental.pallas import tpu as pltpu
from jax.experimental.pallas import tpu_sc as plsc

N_NODES = 10000
HIDDEN = 128
PRED_CH = 64
N_EDGES = 320000

NC = 2
NS = 16
NW = NC * NS
CHUNK = 128
C_PER_W = 79
E_PAD = NW * C_PER_W * CHUNK
N_ACC = 10240
ROWS_PER_TILE = N_ACC // NS

_sc_mesh = plsc.VectorSubcoreMesh(core_axis_name="c", subcore_axis_name="s")


@functools.partial(
    pl.kernel,
    out_type=jax.ShapeDtypeStruct((NC, N_ACC, HIDDEN), jnp.float32),
    mesh=_sc_mesh,
    scratch_types=[
        pltpu.VMEM((C_PER_W, CHUNK), jnp.int32),
        pltpu.VMEM((C_PER_W, CHUNK), jnp.int32),
        pltpu.VMEM((CHUNK, HIDDEN), jnp.float32),
        pltpu.VMEM_SHARED((N_ACC, HIDDEN), jnp.float32),
        pltpu.SemaphoreType.DMA,
    ],
)
def _sc_segment_sum(x_hbm, src_hbm, dst_hbm, zeros_hbm, out_hbm,
                    src_v, dst_v, rows_v, acc, sem):
    cid = lax.axis_index("c")
    sid = lax.axis_index("s")
    wid = sid * NC + cid
    pltpu.sync_copy(zeros_hbm, acc.at[pl.ds(sid * ROWS_PER_TILE, ROWS_PER_TILE)])
    pltpu.sync_copy(src_hbm.at[wid], src_v)
    pltpu.sync_copy(dst_hbm.at[wid], dst_v)
    plsc.subcore_barrier()

    def body(j, carry):
        pltpu.async_copy(x_hbm.at[src_v.at[j]], rows_v, sem).wait()
        pltpu.sync_copy(rows_v, acc.at[dst_v.at[j]], add=True)
        return carry

    lax.fori_loop(0, C_PER_W, body, 0)
    plsc.subcore_barrier()
    pltpu.sync_copy(acc.at[pl.ds(sid * ROWS_PER_TILE, ROWS_PER_TILE)],
                    out_hbm.at[cid, pl.ds(sid * ROWS_PER_TILE, ROWS_PER_TILE)])


BLK = 1000


def _tc_body(p_ref, u_ref, Wh_ref, Wu_ref, b_ref, Wp_ref, bp_ref, xn_ref, y_ref):
    agg = p_ref[0] + p_ref[1]
    h = jnp.dot(agg, Wh_ref[...], preferred_element_type=jnp.float32)
    h = h + jnp.dot(u_ref[...], Wu_ref[...], preferred_element_type=jnp.float32)
    h = h + b_ref[...]
    xn = jnp.maximum(h, 0.0)
    xn_ref[...] = xn
    y_ref[...] = jnp.dot(xn, Wp_ref[...], preferred_element_type=jnp.float32) + bp_ref[...]


_tc_update = pl.pallas_call(
    _tc_body,
    grid=(N_NODES // BLK,),
    in_specs=[
        pl.BlockSpec((NC, BLK, HIDDEN), lambda i: (0, i, 0)),
        pl.BlockSpec((BLK, HIDDEN), lambda i: (i, 0)),
        pl.BlockSpec((HIDDEN, HIDDEN), lambda i: (0, 0)),
        pl.BlockSpec((HIDDEN, HIDDEN), lambda i: (0, 0)),
        pl.BlockSpec((1, HIDDEN), lambda i: (0, 0)),
        pl.BlockSpec((HIDDEN, PRED_CH), lambda i: (0, 0)),
        pl.BlockSpec((1, PRED_CH), lambda i: (0, 0)),
    ],
    out_specs=[
        pl.BlockSpec((BLK, HIDDEN), lambda i: (i, 0)),
        pl.BlockSpec((BLK, PRED_CH), lambda i: (i, 0)),
    ],
    out_shape=[
        jax.ShapeDtypeStruct((N_NODES, HIDDEN), jnp.float32),
        jax.ShapeDtypeStruct((N_NODES, PRED_CH), jnp.float32),
    ],
)


def kernel(x, u, edge_index, W_h, W_u, b, W_p, b_p):
    src = edge_index[0].astype(jnp.int32)
    dst = edge_index[1].astype(jnp.int32)
    pad = E_PAD - N_EDGES
    src = jnp.concatenate([src, jnp.full((pad,), N_NODES, jnp.int32)])
    dst = jnp.concatenate([dst, jnp.zeros((pad,), jnp.int32)])
    src3 = src.reshape(NW, C_PER_W, CHUNK)
    dst3 = dst.reshape(NW, C_PER_W, CHUNK)
    x_pad = jnp.concatenate([x, jnp.zeros((1, HIDDEN), x.dtype)], axis=0)
    zeros_blk = jnp.zeros((ROWS_PER_TILE, HIDDEN), jnp.float32)

    partial = _sc_segment_sum(x_pad, src3, dst3, zeros_blk)

    x_next, y = _tc_update(partial, u, W_h, W_u, b.reshape(1, HIDDEN),
                           W_p, b_p.reshape(1, PRED_CH))
    return (x_next, y)
